# baseline (speedup 1.0000x reference)
import functools

import jax
import jax.numpy as jnp
from jax import lax
from jax.experimental import pallas as pl
from jax.experimental.pallas import tpu as pltpu
from jax.experimental.pallas import tpu_sc as plsc

NC, NS, L = 2, 16, 16
NW = NC * NS
CH = 32


def kernel(x, pos_table):
    B, S, D = x.shape
    seq_per_w = S // NW
    n_chunks = seq_per_w // CH
    chw = CH * D

    xf = x.reshape(B * S * D)
    pf = pos_table.reshape(-1)

    mesh = plsc.VectorSubcoreMesh(core_axis_name="c", subcore_axis_name="s")

    @functools.partial(
        pl.kernel,
        out_type=jax.ShapeDtypeStruct((B * S * D,), jnp.float32),
        mesh=mesh,
        scratch_types=[
            pltpu.VMEM((chw,), jnp.float32),
            pltpu.VMEM((chw,), jnp.float32),
        ],
    )
    def sc_add(x_hbm, p_hbm, o_hbm, xb, pb):
        wid = lax.axis_index("s") * NC + lax.axis_index("c")
        s_base = wid * seq_per_w

        def chunk_body(c, carry):
            s0 = s_base + c * CH
            pltpu.sync_copy(p_hbm.at[pl.ds(s0 * D, chw)], pb)

            def batch_body(b, carry):
                base = (b * S + s0) * D
                pltpu.sync_copy(x_hbm.at[pl.ds(base, chw)], xb)

                def add_body(i, carry):
                    off = pl.multiple_of(i * L, L)
                    xb[pl.ds(off, L)] = xb[pl.ds(off, L)] + pb[pl.ds(off, L)]
                    return carry

                lax.fori_loop(0, chw // L, add_body, 0)
                pltpu.sync_copy(xb, o_hbm.at[pl.ds(base, chw)])
                return carry

            lax.fori_loop(0, B, batch_body, 0)
            return carry

        lax.fori_loop(0, n_chunks, chunk_body, 0)

    return sc_add(xf, pf).reshape(B, S, D)


# speedup vs baseline: 1.6958x; 1.6958x over previous
import functools

import jax
import jax.numpy as jnp
from jax import lax
from jax.experimental import pallas as pl
from jax.experimental.pallas import tpu as pltpu
from jax.experimental.pallas import tpu_sc as plsc

NC, NS, L = 2, 16, 16
NW = NC * NS
CH = 16
NBUF = 4


def kernel(x, pos_table):
    B, S, D = x.shape
    seq_per_w = S // NW
    n_chunks = seq_per_w // CH
    n_items = n_chunks * B
    chw = CH * D

    xf = x.reshape(B * S * D)
    pf = pos_table.reshape(-1)

    mesh = plsc.VectorSubcoreMesh(core_axis_name="c", subcore_axis_name="s")

    @functools.partial(
        pl.kernel,
        out_type=jax.ShapeDtypeStruct((B * S * D,), jnp.float32),
        mesh=mesh,
        scratch_types=(
            [pltpu.VMEM((chw,), jnp.float32) for _ in range(NBUF)]
            + [pltpu.VMEM((chw,), jnp.float32) for _ in range(2)]
            + [pltpu.SemaphoreType.DMA for _ in range(NBUF)]
            + [pltpu.SemaphoreType.DMA for _ in range(NBUF)]
            + [pltpu.SemaphoreType.DMA for _ in range(2)]
        ),
    )
    def sc_add(x_hbm, p_hbm, o_hbm, *refs):
        xb = refs[0:NBUF]
        pb = refs[NBUF:NBUF + 2]
        lsem = refs[NBUF + 2:2 * NBUF + 2]
        ssem = refs[2 * NBUF + 2:3 * NBUF + 2]
        psem = refs[3 * NBUF + 2:3 * NBUF + 4]

        wid = lax.axis_index("s") * NC + lax.axis_index("c")
        s_base = wid * seq_per_w

        def x_base(t):
            c, b = divmod(t, B)
            return (b * S + s_base + c * CH) * D

        def load_x(t):
            p = t % NBUF
            return pltpu.async_copy(
                x_hbm.at[pl.ds(x_base(t), chw)], xb[p], lsem[p])

        def load_pos(c):
            q = c % 2
            return pltpu.async_copy(
                p_hbm.at[pl.ds((s_base + c * CH) * D, chw)], pb[q], psem[q])

        loads = {}
        stores = {}
        posloads = {}

        posloads[0] = load_pos(0)
        if n_chunks > 1:
            posloads[1] = load_pos(1)
        loads[0] = load_x(0)
        loads[1] = load_x(1)

        for t in range(n_items):
            p = t % NBUF
            c, b = divmod(t, B)

            tf = t + 2
            if tf < n_items:
                if tf >= NBUF:
                    stores[tf - NBUF].wait()
                loads[tf] = load_x(tf)
            if b == 0 and c >= 1 and c + 1 < n_chunks:
                posloads[c + 1] = load_pos(c + 1)

            loads[t].wait()
            if b == 0:
                posloads[c].wait()
            xref = xb[p]
            pref = pb[c % 2]

            def add_body(i, xref=xref, pref=pref):
                plsc.addupdate(xref.at[pl.ds(i, L)], pref[pl.ds(i, L)])

            plsc.parallel_loop(0, chw, L, unroll=8)(add_body)

            stores[t] = pltpu.async_copy(
                xb[p], o_hbm.at[pl.ds(x_base(t), chw)], ssem[p])

        for t in range(max(0, n_items - NBUF), n_items):
            stores[t].wait()

    return sc_add(xf, pf).reshape(B, S, D)
